# Initial kernel scaffold; baseline (speedup 1.0000x reference)
import jax
import jax.numpy as jnp
from jax.experimental import pallas as pl


def kernel(x, edge_index, E0):
    raise NotImplementedError("write your pallas kernel here")


# speedup vs baseline: 5.1986x; 5.1986x over previous
import functools

import jax
import jax.numpy as jnp
from jax import lax
from jax.experimental import pallas as pl
from jax.experimental.pallas import tpu as pltpu
from jax.experimental.pallas import tpu_sc as plsc

N = 50000
D = 64
NC = 2
NS = 16
LN = 16

HP = 25088
NP = NC * HP
DUMP = HP
TAB = HP + 256
STR_R = TAB // NS
STR_N = HP // NS
RCH = 32
NRCH = STR_N // RCH

EPT = 51200
EPAD = NS * EPT
BLK = 2048
NBLK = EPT // BLK
CH = 128
NCH = BLK // CH

_mesh = plsc.VectorSubcoreMesh(
    core_axis_name="c", subcore_axis_name="s", num_cores=NC, num_subcores=NS
)


def _dst_local(d, c):
    dl = d - c * HP
    valid = (dl >= 0) & (dl < HP)
    return jnp.where(valid, dl, DUMP)


def _rsqrt16(x):
    i = lax.bitcast_convert_type(x, jnp.int32)
    i = 0x5F3759DF - lax.shift_right_logical(i, 1)
    y = lax.bitcast_convert_type(i, jnp.float32)
    for _ in range(3):
        y = y * (1.5 - 0.5 * x * y * y)
    return jnp.where(x > 0.5, y, 0.0)


def _stage_block(dst_hbm, e_base, dst_blk, loc_blk, c):
    pltpu.sync_copy(dst_hbm.at[pl.ds(e_base, BLK)], dst_blk)

    def body(j, _):
        def inner(v, _):
            d = dst_blk[pl.ds(j * CH + v * LN, LN)]
            loc_blk[j, pl.ds(v * LN, LN)] = _dst_local(d, c)
            return 0

        return lax.fori_loop(0, CH // LN, inner, 0)

    lax.fori_loop(0, NCH, body, 0)


def _prep_body(dst_hbm, e0_hbm, dinv_hbm, u0_hbm,
               deg_s, dinv_s, dst_blk, loc_blk, ones_v, vbuf, rbuf, sem):
    c = lax.axis_index("c")
    s = lax.axis_index("s")

    for v in range(STR_R // LN):
        vbuf[pl.ds(v * LN, LN)] = jnp.zeros((LN,), jnp.float32)
    pltpu.sync_copy(vbuf, deg_s.at[pl.ds(s * STR_R, STR_R)])
    for v in range(CH // LN):
        ones_v[pl.ds(v * LN, LN)] = jnp.ones((LN,), jnp.float32)
    plsc.subcore_barrier()

    def blk_body(b, _):
        e_base = s * EPT + b * BLK
        _stage_block(dst_hbm, e_base, dst_blk, loc_blk, c)

        def ch_body(j, _):
            pltpu.sync_copy(ones_v, deg_s.at[loc_blk.at[j]], add=True)
            return 0

        return lax.fori_loop(0, NCH, ch_body, 0)

    lax.fori_loop(0, NBLK, blk_body, 0)
    plsc.subcore_barrier()

    pltpu.sync_copy(deg_s.at[pl.ds(s * STR_R, STR_R)], vbuf)
    for v in range(STR_R // LN):
        deg = vbuf[pl.ds(v * LN, LN)]
        vbuf[pl.ds(v * LN, LN)] = _rsqrt16(deg)
    pltpu.sync_copy(vbuf, dinv_s.at[pl.ds(s * STR_R, STR_R)])
    plsc.subcore_barrier()

    r0 = s * STR_N
    g0 = c * HP + r0
    pltpu.sync_copy(dinv_s.at[pl.ds(r0, STR_N)], vbuf.at[pl.ds(0, STR_N)])
    pltpu.sync_copy(vbuf.at[pl.ds(0, STR_N)], dinv_hbm.at[pl.ds(g0, STR_N)])

    def wb_body(cc, _):
        pltpu.sync_copy(e0_hbm.at[pl.ds(g0 + cc * RCH, RCH)], rbuf)

        def grp_body(g, _):
            dvec = vbuf[pl.ds(cc * RCH + g * LN, LN)]
            for r in range(LN):
                di = dvec[r]
                row = g * LN + r
                for k in range(D // LN):
                    cs = pl.ds(k * LN, LN)
                    rbuf[row, cs] = rbuf[row, cs] * di
            return 0

        lax.fori_loop(0, RCH // LN, grp_body, 0)
        pltpu.sync_copy(rbuf, u0_hbm.at[pl.ds(g0 + cc * RCH, RCH)])
        return 0

    lax.fori_loop(0, NRCH, wb_body, 0)


def _layer_body(first, last,
                u_hbm, src_hbm, dst_hbm, dinv_hbm, *rest):
    rest = list(rest)
    ssum_in = None if first else rest.pop(0)
    e0_hbm = rest.pop(0) if last else None
    if last:
        emean_hbm = rest.pop(0)
    else:
        unext_hbm = rest.pop(0)
        ssum_out = rest.pop(0)
    (acc_s, src_blk, dst_blk, loc_blk, rows, zbuf, sbuf, ssbuf,
     dvbuf, sem) = rest

    c = lax.axis_index("c")
    s = lax.axis_index("s")

    for r in range(RCH):
        for k in range(D // LN):
            zbuf[r, pl.ds(k * LN, LN)] = jnp.zeros((LN,), jnp.float32)

    def z_body(cc, _):
        pltpu.sync_copy(zbuf, acc_s.at[pl.ds(s * STR_N + cc * RCH, RCH)])
        return 0

    lax.fori_loop(0, NRCH, z_body, 0)

    @pl.when(s == 0)
    def _():
        pltpu.sync_copy(zbuf.at[pl.ds(0, 16)], acc_s.at[pl.ds(HP, 16)])

    plsc.subcore_barrier()

    def blk_body(b, _):
        e_base = s * EPT + b * BLK
        pltpu.sync_copy(src_hbm.at[pl.ds(e_base, BLK)], src_blk)
        _stage_block(dst_hbm, e_base, dst_blk, loc_blk, c)

        def ch_body(j, _):
            idx = src_blk.at[pl.ds(j * CH, CH)]
            pltpu.async_copy(u_hbm.at[idx], rows, sem).wait()
            pltpu.sync_copy(rows, acc_s.at[loc_blk.at[j]], add=True)
            return 0

        return lax.fori_loop(0, NCH, ch_body, 0)

    lax.fori_loop(0, NBLK, blk_body, 0)
    plsc.subcore_barrier()

    def wb_body(cc, _):
        r0 = s * STR_N + cc * RCH
        g0 = c * HP + r0
        pltpu.sync_copy(acc_s.at[pl.ds(r0, RCH)], sbuf)
        pltpu.sync_copy(dinv_hbm.at[pl.ds(g0, RCH)], dvbuf)
        if not first:
            pltpu.sync_copy(ssum_in.at[pl.ds(g0, RCH)], ssbuf)
        if last:
            pltpu.sync_copy(e0_hbm.at[pl.ds(g0, RCH)], zbuf)

        def grp_body(g, _):
            dvec = dvbuf[pl.ds(g * LN, LN)]
            for r in range(LN):
                di = dvec[r]
                row = g * LN + r
                for k in range(D // LN):
                    cs = pl.ds(k * LN, LN)
                    sv = sbuf[row, cs]
                    ss = sv if first else ssbuf[row, cs] + sv
                    if last:
                        zbuf[row, cs] = (zbuf[row, cs] + di * ss) * 0.25
                    else:
                        ssbuf[row, cs] = ss
                        sbuf[row, cs] = sv * (di * di)
            return 0

        lax.fori_loop(0, RCH // LN, grp_body, 0)
        if last:
            pltpu.sync_copy(zbuf, emean_hbm.at[pl.ds(g0, RCH)])
        else:
            pltpu.sync_copy(sbuf, unext_hbm.at[pl.ds(g0, RCH)])
            pltpu.sync_copy(ssbuf, ssum_out.at[pl.ds(g0, RCH)])
        return 0

    lax.fori_loop(0, NRCH, wb_body, 0)


def _f32(*shape):
    return jax.ShapeDtypeStruct(shape, jnp.float32)


_params = pltpu.CompilerParams(use_tc_tiling_on_sc=False)

_prep = pl.kernel(
    _prep_body,
    out_type=(_f32(NP), _f32(NP, D)),
    mesh=_mesh,
    compiler_params=_params,
    scratch_types=[
        pltpu.VMEM_SHARED((TAB,), jnp.float32),
        pltpu.VMEM_SHARED((TAB,), jnp.float32),
        pltpu.VMEM((BLK,), jnp.int32),
        pltpu.VMEM((NCH, CH), jnp.int32),
        pltpu.VMEM((CH,), jnp.float32),
        pltpu.VMEM((STR_R,), jnp.float32),
        pltpu.VMEM((RCH, D), jnp.float32),
        pltpu.SemaphoreType.DMA,
    ],
)


def _make_layer(first, last):
    out_type = (_f32(NP, D),) if last else (_f32(NP, D), _f32(NP, D))
    return pl.kernel(
        functools.partial(_layer_body, first, last),
        out_type=out_type,
        mesh=_mesh,
        compiler_params=_params,
        scratch_types=[
            pltpu.VMEM_SHARED((HP + 16, D), jnp.float32),
            pltpu.VMEM((BLK,), jnp.int32),
            pltpu.VMEM((BLK,), jnp.int32),
            pltpu.VMEM((NCH, CH), jnp.int32),
            pltpu.VMEM((CH, D), jnp.float32),
            pltpu.VMEM((RCH, D), jnp.float32),
            pltpu.VMEM((RCH, D), jnp.float32),
            pltpu.VMEM((RCH, D), jnp.float32),
            pltpu.VMEM((RCH,), jnp.float32),
            pltpu.SemaphoreType.DMA,
        ],
    )


_layer_first = _make_layer(True, False)
_layer_mid = _make_layer(False, False)
_layer_last = _make_layer(False, True)


def kernel(x, edge_index, E0):
    src = edge_index[0]
    dst = edge_index[1]
    e = src.shape[0]
    srcp = jnp.concatenate([src, jnp.zeros((EPAD - e,), jnp.int32)])
    dstp = jnp.concatenate([dst, jnp.full((EPAD - e,), -1, jnp.int32)])
    e0p = jnp.pad(E0, ((0, NP - N), (0, 0)))

    dinv, u0 = _prep(dstp, e0p)
    u1, ss1 = _layer_first(u0, srcp, dstp, dinv)
    u2, ss2 = _layer_mid(u1, srcp, dstp, dinv, ss1)
    (emean_p,) = _layer_last(u2, srcp, dstp, dinv, ss2, e0p)
    return (E0, emean_p[:N])


# speedup vs baseline: 5.4559x; 1.0495x over previous
import functools

import jax
import jax.numpy as jnp
from jax import lax
from jax.experimental import pallas as pl
from jax.experimental.pallas import tpu as pltpu
from jax.experimental.pallas import tpu_sc as plsc

N = 50000
D = 64
NC = 2
NS = 16
LN = 16

HP = 25088
NP = NC * HP
DUMP = HP
TAB = HP + 256
STR_R = TAB // NS
STR_N = HP // NS
RCH = 112
NRCH = STR_N // RCH

EPT = 51200
EPAD = NS * EPT
BLK = 2048
NBLK = EPT // BLK
CH = 128
NCH = BLK // CH

_mesh = plsc.VectorSubcoreMesh(
    core_axis_name="c", subcore_axis_name="s", num_cores=NC, num_subcores=NS
)


def _dst_local(d, c):
    dl = d - c * HP
    valid = (dl >= 0) & (dl < HP)
    return jnp.where(valid, dl, DUMP)


def _rsqrt16(x):
    i = lax.bitcast_convert_type(x, jnp.int32)
    i = 0x5F3759DF - lax.shift_right_logical(i, 1)
    y = lax.bitcast_convert_type(i, jnp.float32)
    for _ in range(3):
        y = y * (1.5 - 0.5 * x * y * y)
    return jnp.where(x > 0.5, y, 0.0)


def _stage_block(dst_hbm, e_base, dst_blk, loc_blk, c):
    pltpu.sync_copy(dst_hbm.at[pl.ds(e_base, BLK)], dst_blk)

    def body(j, _):
        def inner(v, _):
            d = dst_blk[pl.ds(j * CH + v * LN, LN)]
            loc_blk[j, pl.ds(v * LN, LN)] = _dst_local(d, c)
            return 0

        return lax.fori_loop(0, CH // LN, inner, 0)

    lax.fori_loop(0, NCH, body, 0)


def _prep_body(dst_hbm, e0_hbm, dinv_hbm, u0_hbm, loc_hbm,
               deg_s, dinv_s, dst_blk, loc_blk, ones_v, vbuf, rbuf, sem):
    c = lax.axis_index("c")
    s = lax.axis_index("s")

    for v in range(STR_R // LN):
        vbuf[pl.ds(v * LN, LN)] = jnp.zeros((LN,), jnp.float32)
    pltpu.sync_copy(vbuf, deg_s.at[pl.ds(s * STR_R, STR_R)])
    for v in range(CH // LN):
        ones_v[pl.ds(v * LN, LN)] = jnp.ones((LN,), jnp.float32)
    plsc.subcore_barrier()

    def blk_body(b, _):
        e_base = s * EPT + b * BLK
        _stage_block(dst_hbm, e_base, dst_blk, loc_blk, c)
        pltpu.sync_copy(loc_blk, loc_hbm.at[c, pl.ds(e_base // CH, NCH)])
        descs = [
            pltpu.async_copy(ones_v, deg_s.at[loc_blk.at[j]], sem, add=True)
            for j in range(NCH)
        ]
        for d in descs:
            d.wait()
        return 0

    lax.fori_loop(0, NBLK, blk_body, 0)
    plsc.subcore_barrier()

    pltpu.sync_copy(deg_s.at[pl.ds(s * STR_R, STR_R)], vbuf)
    for v in range(STR_R // LN):
        deg = vbuf[pl.ds(v * LN, LN)]
        vbuf[pl.ds(v * LN, LN)] = _rsqrt16(deg)
    pltpu.sync_copy(vbuf, dinv_s.at[pl.ds(s * STR_R, STR_R)])
    plsc.subcore_barrier()

    r0 = s * STR_N
    g0 = c * HP + r0
    pltpu.sync_copy(dinv_s.at[pl.ds(r0, STR_N)], vbuf.at[pl.ds(0, STR_N)])
    pltpu.sync_copy(vbuf.at[pl.ds(0, STR_N)], dinv_hbm.at[pl.ds(g0, STR_N)])

    def wb_body(cc, _):
        pltpu.sync_copy(e0_hbm.at[pl.ds(g0 + cc * RCH, RCH)], rbuf)

        def grp_body(g, _):
            dvec = vbuf[pl.ds(cc * RCH + g * LN, LN)]
            for r in range(LN):
                di = dvec[r]
                row = g * LN + r
                for k in range(D // LN):
                    cs = pl.ds(k * LN, LN)
                    rbuf[row, cs] = rbuf[row, cs] * di
            return 0

        lax.fori_loop(0, RCH // LN, grp_body, 0)
        pltpu.sync_copy(rbuf, u0_hbm.at[pl.ds(g0 + cc * RCH, RCH)])
        return 0

    lax.fori_loop(0, NRCH, wb_body, 0)


KBUF = 3
GAH = 2
SCW = 1


def _layer_body(first, last,
                u_hbm, src_hbm, loc_hbm, dinv_hbm, *rest):
    rest = list(rest)
    ssum_in = None if first else rest.pop(0)
    e0_hbm = rest.pop(0) if last else None
    if last:
        emean_hbm = rest.pop(0)
    else:
        unext_hbm = rest.pop(0)
        ssum_out = rest.pop(0)
    (acc_s, src_blk, loc_blk, rows, dvbuf, sem_g, sem_s) = rest

    c = lax.axis_index("c")
    s = lax.axis_index("s")

    def zrow_body(r, _):
        for k in range(D // LN):
            rows[0, r, pl.ds(k * LN, LN)] = jnp.zeros((LN,), jnp.float32)
        return 0

    lax.fori_loop(0, CH, zrow_body, 0)

    def z_body(cc, _):
        pltpu.sync_copy(rows.at[0, pl.ds(0, RCH)],
                        acc_s.at[pl.ds(s * STR_N + cc * RCH, RCH)])
        return 0

    lax.fori_loop(0, NRCH, z_body, 0)

    @pl.when(s == 0)
    def _():
        pltpu.sync_copy(rows.at[0, pl.ds(0, 16)], acc_s.at[pl.ds(HP, 16)])

    plsc.subcore_barrier()

    def blk_body(b, _):
        e_base = s * EPT + b * BLK
        pltpu.sync_copy(src_hbm.at[pl.ds(e_base, BLK)], src_blk)
        pltpu.sync_copy(loc_hbm.at[c, pl.ds(e_base // CH, NCH)], loc_blk)

        def gather(j):
            idx = src_blk.at[pl.ds(j * CH, CH)]
            return pltpu.async_copy(u_hbm.at[idx], rows.at[j % KBUF], sem_g)

        def scatter(j):
            return pltpu.async_copy(
                rows.at[j % KBUF], acc_s.at[loc_blk.at[j]], sem_s, add=True)

        g_d = [gather(j) for j in range(GAH)]
        s_d = []
        for j in range(NCH):
            if j >= SCW:
                s_d[j - SCW].wait()
            if j + GAH < NCH:
                g_d.append(gather(j + GAH))
            g_d[j].wait()
            s_d.append(scatter(j))
        for j in range(NCH - SCW, NCH):
            s_d[j].wait()
        return 0

    lax.fori_loop(0, NBLK, blk_body, 0)
    plsc.subcore_barrier()

    def wb_body(cc, _):
        r0 = s * STR_N + cc * RCH
        g0 = c * HP + r0
        pltpu.sync_copy(acc_s.at[pl.ds(r0, RCH)], rows.at[0, pl.ds(0, RCH)])
        pltpu.sync_copy(dinv_hbm.at[pl.ds(g0, RCH)], dvbuf.at[pl.ds(0, RCH)])
        if not first:
            pltpu.sync_copy(ssum_in.at[pl.ds(g0, RCH)],
                            rows.at[1, pl.ds(0, RCH)])
        if last:
            pltpu.sync_copy(e0_hbm.at[pl.ds(g0, RCH)],
                            rows.at[2, pl.ds(0, RCH)])

        def grp_body(g, _):
            dvec = dvbuf[pl.ds(g * LN, LN)]
            for r in range(LN):
                di = dvec[r]
                row = g * LN + r
                for k in range(D // LN):
                    cs = pl.ds(k * LN, LN)
                    sv = rows[0, row, cs]
                    ss = sv if first else rows[1, row, cs] + sv
                    if last:
                        rows[2, row, cs] = (rows[2, row, cs] + di * ss) * 0.25
                    else:
                        rows[1, row, cs] = ss
                        rows[0, row, cs] = sv * (di * di)
            return 0

        lax.fori_loop(0, RCH // LN, grp_body, 0)
        if last:
            pltpu.sync_copy(rows.at[2, pl.ds(0, RCH)],
                            emean_hbm.at[pl.ds(g0, RCH)])
        else:
            pltpu.sync_copy(rows.at[0, pl.ds(0, RCH)],
                            unext_hbm.at[pl.ds(g0, RCH)])
            pltpu.sync_copy(rows.at[1, pl.ds(0, RCH)],
                            ssum_out.at[pl.ds(g0, RCH)])
        return 0

    lax.fori_loop(0, NRCH, wb_body, 0)


def _f32(*shape):
    return jax.ShapeDtypeStruct(shape, jnp.float32)


_params = pltpu.CompilerParams(use_tc_tiling_on_sc=False)

_prep = pl.kernel(
    _prep_body,
    out_type=(
        _f32(NP),
        _f32(NP, D),
        jax.ShapeDtypeStruct((NC, EPAD // CH, CH), jnp.int32),
    ),
    mesh=_mesh,
    compiler_params=_params,
    scratch_types=[
        pltpu.VMEM_SHARED((TAB,), jnp.float32),
        pltpu.VMEM_SHARED((TAB,), jnp.float32),
        pltpu.VMEM((BLK,), jnp.int32),
        pltpu.VMEM((NCH, CH), jnp.int32),
        pltpu.VMEM((CH,), jnp.float32),
        pltpu.VMEM((STR_R,), jnp.float32),
        pltpu.VMEM((RCH, D), jnp.float32),
        pltpu.SemaphoreType.DMA,
    ],
)


def _make_layer(first, last):
    out_type = (_f32(NP, D),) if last else (_f32(NP, D), _f32(NP, D))
    return pl.kernel(
        functools.partial(_layer_body, first, last),
        out_type=out_type,
        mesh=_mesh,
        compiler_params=_params,
        scratch_types=[
            pltpu.VMEM_SHARED((HP + 16, D), jnp.float32),
            pltpu.VMEM((BLK,), jnp.int32),
            pltpu.VMEM((NCH, CH), jnp.int32),
            pltpu.VMEM((KBUF, CH, D), jnp.float32),
            pltpu.VMEM((CH,), jnp.float32),
            pltpu.SemaphoreType.DMA,
            pltpu.SemaphoreType.DMA,
        ],
    )


_layer_first = _make_layer(True, False)
_layer_mid = _make_layer(False, False)
_layer_last = _make_layer(False, True)


def kernel(x, edge_index, E0):
    src = edge_index[0]
    dst = edge_index[1]
    e = src.shape[0]
    srcp = jnp.concatenate([src, jnp.zeros((EPAD - e,), jnp.int32)])
    dstp = jnp.concatenate([dst, jnp.full((EPAD - e,), -1, jnp.int32)])
    e0p = jnp.pad(E0, ((0, NP - N), (0, 0)))

    dinv, u0, loc = _prep(dstp, e0p)
    u1, ss1 = _layer_first(u0, srcp, loc, dinv)
    u2, ss2 = _layer_mid(u1, srcp, loc, dinv, ss1)
    (emean_p,) = _layer_last(u2, srcp, loc, dinv, ss2, e0p)
    return (E0, emean_p[:N])


# speedup vs baseline: 7.5270x; 1.3796x over previous
import functools

import jax
import jax.numpy as jnp
from jax import lax
from jax.experimental import pallas as pl
from jax.experimental.pallas import tpu as pltpu
from jax.experimental.pallas import tpu_sc as plsc

N = 50000
D = 64
NC = 2
NS = 16
LN = 16

HP = 25088
NP = NC * HP
DUMP = HP
TAB = HP + 256
STR_R = TAB // NS
STR_N = HP // NS
RCH = 112
NRCH = STR_N // RCH

EPT = 51200
EPAD = NS * EPT
BLK = 2048
NBLK = EPT // BLK
CH = 128
NCH = BLK // CH
LCAP = EPT + BLK

_mesh = plsc.VectorSubcoreMesh(
    core_axis_name="c", subcore_axis_name="s", num_cores=NC, num_subcores=NS
)


def _rsqrt16(x):
    i = lax.bitcast_convert_type(x, jnp.int32)
    i = 0x5F3759DF - lax.shift_right_logical(i, 1)
    y = lax.bitcast_convert_type(i, jnp.float32)
    for _ in range(3):
        y = y * (1.5 - 0.5 * x * y * y)
    return jnp.where(x > 0.5, y, 0.0)


def _prep_body(src_hbm, dst_hbm, e0_hbm,
               dinv_hbm, u0_hbm, srcl_hbm, locl_hbm, cnt_hbm,
               deg_s, dinv_s, src_sp, loc_sp, src_blk, dst_blk,
               idx_buf, ldl_buf, zpad, dpad, ones_v, cvec, pfx,
               vbuf, rbuf, sem):
    c = lax.axis_index("c")
    s = lax.axis_index("s")

    for v in range(STR_R // LN):
        vbuf[pl.ds(v * LN, LN)] = jnp.zeros((LN,), jnp.float32)
    pltpu.sync_copy(vbuf, deg_s.at[pl.ds(s * STR_R, STR_R)])
    for v in range(CH // LN):
        ones_v[pl.ds(v * LN, LN)] = jnp.ones((LN,), jnp.float32)
    plsc.subcore_barrier()

    lo = c * HP
    base = s * LCAP
    iota = lax.iota(jnp.int32, LN)
    for v in range(3):
        pfx[pl.ds(v * LN, LN)] = jnp.zeros((LN,), jnp.int32)

    def blk_body(b, pos):
        e_base = s * EPT + b * BLK
        pltpu.sync_copy(src_hbm.at[pl.ds(e_base, BLK)], src_blk)
        pltpu.sync_copy(dst_hbm.at[pl.ds(e_base, BLK)], dst_blk)

        descs = []
        for ch in range(NCH):
            if ch >= 4:
                for dsc in descs[3 * (ch - 4):3 * (ch - 3)]:
                    dsc.wait()

            def grp(g, pos, ch=ch):
                d = dst_blk[pl.ds(ch * CH + g * LN, LN)]
                dl = d - lo
                m = (dl >= 0) & (dl < HP)
                mi = jnp.where(m, jnp.int32(1), jnp.int32(0))
                x = mi
                for k in (1, 2, 4, 8):
                    pfx[pl.ds(LN, LN)] = x
                    x = x + pfx[pl.ds(LN - k, LN)]
                y = mi
                for k in (1, 2, 4, 8):
                    pfx[pl.ds(LN, LN)] = y
                    y = y + pfx[pl.ds(LN + k, LN)]
                tot = x + y - mi
                idx_buf[ch, pl.ds(g * LN, LN)] = jnp.where(
                    m, base + (pos + x - 1), base + (LCAP - 1))
                ldl_buf[ch, pl.ds(g * LN, LN)] = jnp.where(m, dl, DUMP)
                return pos + tot

            pos = lax.fori_loop(0, CH // LN, grp, pos)
            descs.append(pltpu.async_copy(
                src_blk.at[pl.ds(ch * CH, CH)],
                src_sp.at[idx_buf.at[ch]], sem))
            descs.append(pltpu.async_copy(
                ldl_buf.at[ch],
                loc_sp.at[idx_buf.at[ch]], sem))
            descs.append(pltpu.async_copy(
                ones_v, deg_s.at[ldl_buf.at[ch]], sem, add=True))
        for dsc in descs[3 * (NCH - 4):]:
            dsc.wait()
        return pos

    pos = lax.fori_loop(0, NBLK, blk_body, jnp.zeros((LN,), jnp.int32))
    cnt = pos[0]
    cntp_v = jnp.bitwise_and(pos + (BLK - 1), ~jnp.int32(BLK - 1))

    for v in range(CH // LN):
        zpad[pl.ds(v * LN, LN)] = jnp.zeros((LN,), jnp.int32)
        dpad[pl.ds(v * LN, LN)] = jnp.full((LN,), DUMP, jnp.int32)

    descs = []
    for j in range(NCH):
        if j >= 4:
            for dsc in descs[2 * (j - 4):2 * (j - 3)]:
                dsc.wait()
        for v in range(CH // LN):
            idx_buf[j, pl.ds(v * LN, LN)] = (
                base + cnt + (j * CH + v * LN) + iota)
        descs.append(pltpu.async_copy(
            zpad, src_sp.at[idx_buf.at[j]], sem))
        descs.append(pltpu.async_copy(
            dpad, loc_sp.at[idx_buf.at[j]], sem))
    for dsc in descs[2 * (NCH - 4):]:
        dsc.wait()

    cvec[pl.ds(0, LN)] = cntp_v
    pltpu.sync_copy(cvec, cnt_hbm.at[c, s])
    pltpu.sync_copy(src_sp.at[pl.ds(base, EPT)], srcl_hbm.at[c, s])
    pltpu.sync_copy(loc_sp.at[pl.ds(base, EPT)], locl_hbm.at[c, s])
    plsc.subcore_barrier()

    pltpu.sync_copy(deg_s.at[pl.ds(s * STR_R, STR_R)], vbuf)
    for v in range(STR_R // LN):
        deg = vbuf[pl.ds(v * LN, LN)]
        vbuf[pl.ds(v * LN, LN)] = _rsqrt16(deg)
    pltpu.sync_copy(vbuf, dinv_s.at[pl.ds(s * STR_R, STR_R)])
    plsc.subcore_barrier()

    r0 = s * STR_N
    g0 = c * HP + r0
    pltpu.sync_copy(dinv_s.at[pl.ds(r0, STR_N)], vbuf.at[pl.ds(0, STR_N)])
    pltpu.sync_copy(vbuf.at[pl.ds(0, STR_N)], dinv_hbm.at[pl.ds(g0, STR_N)])

    def wb_body(cc, _):
        pltpu.sync_copy(e0_hbm.at[pl.ds(g0 + cc * RCH, RCH)], rbuf)

        def grp_body(g, _):
            dvec = vbuf[pl.ds(cc * RCH + g * LN, LN)]
            for r in range(LN):
                di = dvec[r]
                row = g * LN + r
                for k in range(D // LN):
                    cs = pl.ds(k * LN, LN)
                    rbuf[row, cs] = rbuf[row, cs] * di
            return 0

        lax.fori_loop(0, RCH // LN, grp_body, 0)
        pltpu.sync_copy(rbuf, u0_hbm.at[pl.ds(g0 + cc * RCH, RCH)])
        return 0

    lax.fori_loop(0, NRCH, wb_body, 0)


KBUF = 3
GAH = 2
SCW = 1


def _layer_body(first, last,
                u_hbm, srcl_hbm, locl_hbm, cnt_hbm, dinv_hbm, *rest):
    rest = list(rest)
    ssum_in = None if first else rest.pop(0)
    e0_hbm = rest.pop(0) if last else None
    if last:
        emean_hbm = rest.pop(0)
    else:
        unext_hbm = rest.pop(0)
        ssum_out = rest.pop(0)
    (acc_s, src_blk, loc_blk, cvec, rows, dvbuf, sem_g, sem_s) = rest

    c = lax.axis_index("c")
    s = lax.axis_index("s")

    def zrow_body(r, _):
        for k in range(D // LN):
            rows[0, r, pl.ds(k * LN, LN)] = jnp.zeros((LN,), jnp.float32)
        return 0

    lax.fori_loop(0, CH, zrow_body, 0)

    def z_body(cc, _):
        pltpu.sync_copy(rows.at[0, pl.ds(0, RCH)],
                        acc_s.at[pl.ds(s * STR_N + cc * RCH, RCH)])
        return 0

    lax.fori_loop(0, NRCH, z_body, 0)

    @pl.when(s == 0)
    def _():
        pltpu.sync_copy(rows.at[0, pl.ds(0, 16)], acc_s.at[pl.ds(HP, 16)])

    pltpu.sync_copy(cnt_hbm.at[c, s], cvec)
    cntp = cvec[pl.ds(0, LN)][0]
    plsc.subcore_barrier()

    def blk_body(b, _):
        @pl.when(b * BLK < cntp)
        def _():
            pltpu.sync_copy(srcl_hbm.at[c, s, pl.ds(b * BLK, BLK)], src_blk)
            pltpu.sync_copy(locl_hbm.at[c, s, pl.ds(b * BLK, BLK)], loc_blk)

            def gather(j):
                idx = src_blk.at[pl.ds(j * CH, CH)]
                return pltpu.async_copy(u_hbm.at[idx], rows.at[j % KBUF],
                                        sem_g)

            def scatter(j):
                idx = loc_blk.at[pl.ds(j * CH, CH)]
                return pltpu.async_copy(
                    rows.at[j % KBUF], acc_s.at[idx], sem_s, add=True)

            g_d = [gather(j) for j in range(GAH)]
            s_d = []
            for j in range(NCH):
                if j >= SCW:
                    s_d[j - SCW].wait()
                if j + GAH < NCH:
                    g_d.append(gather(j + GAH))
                g_d[j].wait()
                s_d.append(scatter(j))
            for j in range(NCH - SCW, NCH):
                s_d[j].wait()

        return 0

    lax.fori_loop(0, NBLK, blk_body, 0)
    plsc.subcore_barrier()

    def wb_body(cc, _):
        r0 = s * STR_N + cc * RCH
        g0 = c * HP + r0
        pltpu.sync_copy(acc_s.at[pl.ds(r0, RCH)], rows.at[0, pl.ds(0, RCH)])
        pltpu.sync_copy(dinv_hbm.at[pl.ds(g0, RCH)], dvbuf.at[pl.ds(0, RCH)])
        if not first:
            pltpu.sync_copy(ssum_in.at[pl.ds(g0, RCH)],
                            rows.at[1, pl.ds(0, RCH)])
        if last:
            pltpu.sync_copy(e0_hbm.at[pl.ds(g0, RCH)],
                            rows.at[2, pl.ds(0, RCH)])

        def grp_body(g, _):
            dvec = dvbuf[pl.ds(g * LN, LN)]
            for r in range(LN):
                di = dvec[r]
                row = g * LN + r
                for k in range(D // LN):
                    cs = pl.ds(k * LN, LN)
                    sv = rows[0, row, cs]
                    ss = sv if first else rows[1, row, cs] + sv
                    if last:
                        rows[2, row, cs] = (rows[2, row, cs] + di * ss) * 0.25
                    else:
                        rows[1, row, cs] = ss
                        rows[0, row, cs] = sv * (di * di)
            return 0

        lax.fori_loop(0, RCH // LN, grp_body, 0)
        if last:
            pltpu.sync_copy(rows.at[2, pl.ds(0, RCH)],
                            emean_hbm.at[pl.ds(g0, RCH)])
        else:
            pltpu.sync_copy(rows.at[0, pl.ds(0, RCH)],
                            unext_hbm.at[pl.ds(g0, RCH)])
            pltpu.sync_copy(rows.at[1, pl.ds(0, RCH)],
                            ssum_out.at[pl.ds(g0, RCH)])
        return 0

    lax.fori_loop(0, NRCH, wb_body, 0)


def _f32(*shape):
    return jax.ShapeDtypeStruct(shape, jnp.float32)


def _i32(*shape):
    return jax.ShapeDtypeStruct(shape, jnp.int32)


_params = pltpu.CompilerParams(use_tc_tiling_on_sc=False)

_prep = pl.kernel(
    _prep_body,
    out_type=(
        _f32(NP),
        _f32(NP, D),
        _i32(NC, NS, EPT),
        _i32(NC, NS, EPT),
        _i32(NC, NS, LN),
    ),
    mesh=_mesh,
    compiler_params=_params,
    scratch_types=[
        pltpu.VMEM_SHARED((TAB,), jnp.float32),
        pltpu.VMEM_SHARED((TAB,), jnp.float32),
        pltpu.VMEM_SHARED((NS * LCAP,), jnp.int32),
        pltpu.VMEM_SHARED((NS * LCAP,), jnp.int32),
        pltpu.VMEM((BLK,), jnp.int32),
        pltpu.VMEM((BLK,), jnp.int32),
        pltpu.VMEM((NCH, CH), jnp.int32),
        pltpu.VMEM((NCH, CH), jnp.int32),
        pltpu.VMEM((CH,), jnp.int32),
        pltpu.VMEM((CH,), jnp.int32),
        pltpu.VMEM((CH,), jnp.float32),
        pltpu.VMEM((LN,), jnp.int32),
        pltpu.VMEM((3 * LN,), jnp.int32),
        pltpu.VMEM((STR_R,), jnp.float32),
        pltpu.VMEM((RCH, D), jnp.float32),
        pltpu.SemaphoreType.DMA,
    ],
)


def _make_layer(first, last):
    out_type = (_f32(NP, D),) if last else (_f32(NP, D), _f32(NP, D))
    return pl.kernel(
        functools.partial(_layer_body, first, last),
        out_type=out_type,
        mesh=_mesh,
        compiler_params=_params,
        scratch_types=[
            pltpu.VMEM_SHARED((HP + 16, D), jnp.float32),
            pltpu.VMEM((BLK,), jnp.int32),
            pltpu.VMEM((BLK,), jnp.int32),
            pltpu.VMEM((LN,), jnp.int32),
            pltpu.VMEM((KBUF, CH, D), jnp.float32),
            pltpu.VMEM((CH,), jnp.float32),
            pltpu.SemaphoreType.DMA,
            pltpu.SemaphoreType.DMA,
        ],
    )


_layer_first = _make_layer(True, False)
_layer_mid = _make_layer(False, False)
_layer_last = _make_layer(False, True)


def kernel(x, edge_index, E0):
    src = edge_index[0]
    dst = edge_index[1]
    e = src.shape[0]
    srcp = jnp.concatenate([src, jnp.zeros((EPAD - e,), jnp.int32)])
    dstp = jnp.concatenate([dst, jnp.full((EPAD - e,), -1, jnp.int32)])
    e0p = jnp.pad(E0, ((0, NP - N), (0, 0)))

    dinv, u0, srcl, locl, cnt = _prep(srcp, dstp, e0p)
    u1, ss1 = _layer_first(u0, srcl, locl, cnt, dinv)
    u2, ss2 = _layer_mid(u1, srcl, locl, cnt, dinv, ss1)
    (emean_p,) = _layer_last(u2, srcl, locl, cnt, dinv, ss2, e0p)
    return (E0, emean_p[:N])


# speedup vs baseline: 7.5457x; 1.0025x over previous
import functools

import jax
import jax.numpy as jnp
from jax import lax
from jax.experimental import pallas as pl
from jax.experimental.pallas import tpu as pltpu
from jax.experimental.pallas import tpu_sc as plsc

N = 50000
D = 64
NC = 2
NS = 16
LN = 16

HP = 25088
NP = NC * HP
DUMP = HP
TAB = HP + 256
STR_R = TAB // NS
STR_N = HP // NS
RCH = 112
NRCH = STR_N // RCH

EPT = 51200
EPAD = NS * EPT
BLK = 2048
NBLK = EPT // BLK
CH = 128
NCH = BLK // CH
LCAP = EPT + BLK

_mesh = plsc.VectorSubcoreMesh(
    core_axis_name="c", subcore_axis_name="s", num_cores=NC, num_subcores=NS
)


def _rsqrt16(x):
    i = lax.bitcast_convert_type(x, jnp.int32)
    i = 0x5F3759DF - lax.shift_right_logical(i, 1)
    y = lax.bitcast_convert_type(i, jnp.float32)
    for _ in range(3):
        y = y * (1.5 - 0.5 * x * y * y)
    return jnp.where(x > 0.5, y, 0.0)


def _prep_body(src_hbm, dst_hbm, e0_hbm,
               dinv_hbm, u0_hbm, srcl_hbm, locl_hbm, cnt_hbm,
               deg_s, dinv_s, src_sp, loc_sp, src_blk, dst_blk,
               idx_buf, ldl_buf, zpad, dpad, ones_v, cvec, pfx,
               vbuf, rbuf, sem):
    c = lax.axis_index("c")
    s = lax.axis_index("s")

    for v in range(STR_R // LN):
        vbuf[pl.ds(v * LN, LN)] = jnp.zeros((LN,), jnp.float32)
    pltpu.sync_copy(vbuf, deg_s.at[pl.ds(s * STR_R, STR_R)])
    for v in range(CH // LN):
        ones_v[pl.ds(v * LN, LN)] = jnp.ones((LN,), jnp.float32)
    plsc.subcore_barrier()

    lo = c * HP
    base = s * LCAP
    iota = lax.iota(jnp.int32, LN)
    for v in range(3):
        pfx[pl.ds(v * LN, LN)] = jnp.zeros((LN,), jnp.int32)

    def blk_body(b, pos):
        e_base = s * EPT + b * BLK
        pltpu.sync_copy(src_hbm.at[pl.ds(e_base, BLK)], src_blk)
        pltpu.sync_copy(dst_hbm.at[pl.ds(e_base, BLK)], dst_blk)

        descs = []
        for ch in range(NCH):
            if ch >= 4:
                for dsc in descs[3 * (ch - 4):3 * (ch - 3)]:
                    dsc.wait()

            def grp(g, pos, ch=ch):
                d = dst_blk[pl.ds(ch * CH + g * LN, LN)]
                dl = d - lo
                m = (dl >= 0) & (dl < HP)
                mi = jnp.where(m, jnp.int32(1), jnp.int32(0))
                x = mi
                for k in (1, 2, 4, 8):
                    pfx[pl.ds(LN, LN)] = x
                    x = x + pfx[pl.ds(LN - k, LN)]
                y = mi
                for k in (1, 2, 4, 8):
                    pfx[pl.ds(LN, LN)] = y
                    y = y + pfx[pl.ds(LN + k, LN)]
                tot = x + y - mi
                idx_buf[ch, pl.ds(g * LN, LN)] = jnp.where(
                    m, base + (pos + x - 1), base + (LCAP - 1))
                ldl_buf[ch, pl.ds(g * LN, LN)] = jnp.where(m, dl, DUMP)
                return pos + tot

            pos = lax.fori_loop(0, CH // LN, grp, pos)
            descs.append(pltpu.async_copy(
                src_blk.at[pl.ds(ch * CH, CH)],
                src_sp.at[idx_buf.at[ch]], sem))
            descs.append(pltpu.async_copy(
                ldl_buf.at[ch],
                loc_sp.at[idx_buf.at[ch]], sem))
            descs.append(pltpu.async_copy(
                ones_v, deg_s.at[ldl_buf.at[ch]], sem, add=True))
        for dsc in descs[3 * (NCH - 4):]:
            dsc.wait()
        return pos

    pos = lax.fori_loop(0, NBLK, blk_body, jnp.zeros((LN,), jnp.int32))
    cnt = pos[0]
    cntp_v = jnp.bitwise_and(pos + (BLK - 1), ~jnp.int32(BLK - 1))

    for v in range(CH // LN):
        zpad[pl.ds(v * LN, LN)] = jnp.zeros((LN,), jnp.int32)
        dpad[pl.ds(v * LN, LN)] = jnp.full((LN,), DUMP, jnp.int32)

    descs = []
    for j in range(NCH):
        if j >= 4:
            for dsc in descs[2 * (j - 4):2 * (j - 3)]:
                dsc.wait()
        for v in range(CH // LN):
            idx_buf[j, pl.ds(v * LN, LN)] = (
                base + cnt + (j * CH + v * LN) + iota)
        descs.append(pltpu.async_copy(
            zpad, src_sp.at[idx_buf.at[j]], sem))
        descs.append(pltpu.async_copy(
            dpad, loc_sp.at[idx_buf.at[j]], sem))
    for dsc in descs[2 * (NCH - 4):]:
        dsc.wait()

    cvec[pl.ds(0, LN)] = cntp_v
    pltpu.sync_copy(cvec, cnt_hbm.at[c, s])
    pltpu.sync_copy(src_sp.at[pl.ds(base, EPT)], srcl_hbm.at[c, s])
    pltpu.sync_copy(loc_sp.at[pl.ds(base, EPT)], locl_hbm.at[c, s])
    plsc.subcore_barrier()

    pltpu.sync_copy(deg_s.at[pl.ds(s * STR_R, STR_R)], vbuf)
    for v in range(STR_R // LN):
        deg = vbuf[pl.ds(v * LN, LN)]
        vbuf[pl.ds(v * LN, LN)] = _rsqrt16(deg)
    pltpu.sync_copy(vbuf, dinv_s.at[pl.ds(s * STR_R, STR_R)])
    plsc.subcore_barrier()

    r0 = s * STR_N
    g0 = c * HP + r0
    pltpu.sync_copy(dinv_s.at[pl.ds(r0, STR_N)], vbuf.at[pl.ds(0, STR_N)])
    pltpu.sync_copy(vbuf.at[pl.ds(0, STR_N)], dinv_hbm.at[pl.ds(g0, STR_N)])

    def wb_body(cc, _):
        pltpu.sync_copy(e0_hbm.at[pl.ds(g0 + cc * RCH, RCH)], rbuf)

        def grp_body(g, _):
            dvec = vbuf[pl.ds(cc * RCH + g * LN, LN)]
            for r in range(LN):
                di = dvec[r]
                row = g * LN + r
                for k in range(D // LN):
                    cs = pl.ds(k * LN, LN)
                    rbuf[row, cs] = rbuf[row, cs] * di
            return 0

        lax.fori_loop(0, RCH // LN, grp_body, 0)
        pltpu.sync_copy(rbuf, u0_hbm.at[pl.ds(g0 + cc * RCH, RCH)])
        return 0

    lax.fori_loop(0, NRCH, wb_body, 0)


KBUF = 3
GAH = 2
SCW = 1
LBLK = 1024
LNCH = LBLK // CH
LNBLK = EPT // LBLK


def _layer_body(first, last,
                u_hbm, srcl_hbm, locl_hbm, cnt_hbm, dinv_hbm, *rest):
    rest = list(rest)
    ssum_in = None if first else rest.pop(0)
    e0_hbm = rest.pop(0) if last else None
    if last:
        emean_hbm = rest.pop(0)
    else:
        unext_hbm = rest.pop(0)
        ssum_out = rest.pop(0)
    (acc_s, src_blk, loc_blk, cvec, rows, dvbuf, sem_g, sem_s, sem_e) = rest

    c = lax.axis_index("c")
    s = lax.axis_index("s")

    def zrow_body(r, _):
        for k in range(D // LN):
            rows[0, r, pl.ds(k * LN, LN)] = jnp.zeros((LN,), jnp.float32)
        return 0

    lax.fori_loop(0, CH, zrow_body, 0)

    def z_body(cc, _):
        pltpu.sync_copy(rows.at[0, pl.ds(0, RCH)],
                        acc_s.at[pl.ds(s * STR_N + cc * RCH, RCH)])
        return 0

    lax.fori_loop(0, NRCH, z_body, 0)

    @pl.when(s == 0)
    def _():
        pltpu.sync_copy(rows.at[0, pl.ds(0, 16)], acc_s.at[pl.ds(HP, 16)])

    pltpu.sync_copy(cnt_hbm.at[c, s], cvec)
    cntp = cvec[pl.ds(0, LN)][0]
    plsc.subcore_barrier()

    def eload(b, slot):
        pltpu.async_copy(srcl_hbm.at[c, s, pl.ds(b * LBLK, LBLK)],
                         src_blk.at[slot], sem_e)
        pltpu.async_copy(locl_hbm.at[c, s, pl.ds(b * LBLK, LBLK)],
                         loc_blk.at[slot], sem_e)

    def ewait(slot):
        pltpu.make_async_copy(srcl_hbm.at[c, s, pl.ds(0, LBLK)],
                              src_blk.at[slot], sem_e).wait()
        pltpu.make_async_copy(locl_hbm.at[c, s, pl.ds(0, LBLK)],
                              loc_blk.at[slot], sem_e).wait()

    def process(b, slot, bnext):
        ewait(slot)

        @pl.when(bnext * LBLK < cntp)
        def _():
            eload(bnext, 1 - slot)

        def gather(j):
            idx = src_blk.at[slot, pl.ds(j * CH, CH)]
            return pltpu.async_copy(u_hbm.at[idx], rows.at[j % KBUF], sem_g)

        def scatter(j):
            idx = loc_blk.at[slot, pl.ds(j * CH, CH)]
            return pltpu.async_copy(
                rows.at[j % KBUF], acc_s.at[idx], sem_s, add=True)

        g_d = [gather(j) for j in range(GAH)]
        s_d = []
        for j in range(LNCH):
            if j >= SCW:
                s_d[j - SCW].wait()
            if j + GAH < LNCH:
                g_d.append(gather(j + GAH))
            g_d[j].wait()
            s_d.append(scatter(j))
        for j in range(LNCH - SCW, LNCH):
            s_d[j].wait()

    @pl.when(0 < cntp)
    def _():
        eload(0, 0)

    def pair_body(i, _):
        b0 = 2 * i
        b1 = 2 * i + 1

        @pl.when(b0 * LBLK < cntp)
        def _():
            process(b0, 0, b1)

        @pl.when(b1 * LBLK < cntp)
        def _():
            process(b1, 1, b1 + 1)

        return 0

    lax.fori_loop(0, (LNBLK + 1) // 2, pair_body, 0)
    plsc.subcore_barrier()

    def wb_body(cc, _):
        r0 = s * STR_N + cc * RCH
        g0 = c * HP + r0
        pltpu.sync_copy(acc_s.at[pl.ds(r0, RCH)], rows.at[0, pl.ds(0, RCH)])
        pltpu.sync_copy(dinv_hbm.at[pl.ds(g0, RCH)], dvbuf.at[pl.ds(0, RCH)])
        if not first:
            pltpu.sync_copy(ssum_in.at[pl.ds(g0, RCH)],
                            rows.at[1, pl.ds(0, RCH)])
        if last:
            pltpu.sync_copy(e0_hbm.at[pl.ds(g0, RCH)],
                            rows.at[2, pl.ds(0, RCH)])

        def grp_body(g, _):
            dvec = dvbuf[pl.ds(g * LN, LN)]
            for r in range(LN):
                di = dvec[r]
                row = g * LN + r
                for k in range(D // LN):
                    cs = pl.ds(k * LN, LN)
                    sv = rows[0, row, cs]
                    ss = sv if first else rows[1, row, cs] + sv
                    if last:
                        rows[2, row, cs] = (rows[2, row, cs] + di * ss) * 0.25
                    else:
                        rows[1, row, cs] = ss
                        rows[0, row, cs] = sv * (di * di)
            return 0

        lax.fori_loop(0, RCH // LN, grp_body, 0)
        if last:
            pltpu.sync_copy(rows.at[2, pl.ds(0, RCH)],
                            emean_hbm.at[pl.ds(g0, RCH)])
        else:
            pltpu.sync_copy(rows.at[0, pl.ds(0, RCH)],
                            unext_hbm.at[pl.ds(g0, RCH)])
            pltpu.sync_copy(rows.at[1, pl.ds(0, RCH)],
                            ssum_out.at[pl.ds(g0, RCH)])
        return 0

    lax.fori_loop(0, NRCH, wb_body, 0)


def _f32(*shape):
    return jax.ShapeDtypeStruct(shape, jnp.float32)


def _i32(*shape):
    return jax.ShapeDtypeStruct(shape, jnp.int32)


_params = pltpu.CompilerParams(use_tc_tiling_on_sc=False)

_prep = pl.kernel(
    _prep_body,
    out_type=(
        _f32(NP),
        _f32(NP, D),
        _i32(NC, NS, EPT),
        _i32(NC, NS, EPT),
        _i32(NC, NS, LN),
    ),
    mesh=_mesh,
    compiler_params=_params,
    scratch_types=[
        pltpu.VMEM_SHARED((TAB,), jnp.float32),
        pltpu.VMEM_SHARED((TAB,), jnp.float32),
        pltpu.VMEM_SHARED((NS * LCAP,), jnp.int32),
        pltpu.VMEM_SHARED((NS * LCAP,), jnp.int32),
        pltpu.VMEM((BLK,), jnp.int32),
        pltpu.VMEM((BLK,), jnp.int32),
        pltpu.VMEM((NCH, CH), jnp.int32),
        pltpu.VMEM((NCH, CH), jnp.int32),
        pltpu.VMEM((CH,), jnp.int32),
        pltpu.VMEM((CH,), jnp.int32),
        pltpu.VMEM((CH,), jnp.float32),
        pltpu.VMEM((LN,), jnp.int32),
        pltpu.VMEM((3 * LN,), jnp.int32),
        pltpu.VMEM((STR_R,), jnp.float32),
        pltpu.VMEM((RCH, D), jnp.float32),
        pltpu.SemaphoreType.DMA,
    ],
)


def _make_layer(first, last):
    out_type = (_f32(NP, D),) if last else (_f32(NP, D), _f32(NP, D))
    return pl.kernel(
        functools.partial(_layer_body, first, last),
        out_type=out_type,
        mesh=_mesh,
        compiler_params=_params,
        scratch_types=[
            pltpu.VMEM_SHARED((HP + 16, D), jnp.float32),
            pltpu.VMEM((2, LBLK), jnp.int32),
            pltpu.VMEM((2, LBLK), jnp.int32),
            pltpu.VMEM((LN,), jnp.int32),
            pltpu.VMEM((KBUF, CH, D), jnp.float32),
            pltpu.VMEM((CH,), jnp.float32),
            pltpu.SemaphoreType.DMA,
            pltpu.SemaphoreType.DMA,
            pltpu.SemaphoreType.DMA,
        ],
    )


_layer_first = _make_layer(True, False)
_layer_mid = _make_layer(False, False)
_layer_last = _make_layer(False, True)


def kernel(x, edge_index, E0):
    src = edge_index[0]
    dst = edge_index[1]
    e = src.shape[0]
    srcp = jnp.concatenate([src, jnp.zeros((EPAD - e,), jnp.int32)])
    dstp = jnp.concatenate([dst, jnp.full((EPAD - e,), -1, jnp.int32)])
    e0p = jnp.pad(E0, ((0, NP - N), (0, 0)))

    dinv, u0, srcl, locl, cnt = _prep(srcp, dstp, e0p)
    u1, ss1 = _layer_first(u0, srcl, locl, cnt, dinv)
    u2, ss2 = _layer_mid(u1, srcl, locl, cnt, dinv, ss1)
    (emean_p,) = _layer_last(u2, srcl, locl, cnt, dinv, ss2, e0p)
    return (E0, emean_p[:N])


# speedup vs baseline: 10.5467x; 1.3977x over previous
import functools

import jax
import jax.numpy as jnp
from jax import lax
from jax.experimental import pallas as pl
from jax.experimental.pallas import tpu as pltpu
from jax.experimental.pallas import tpu_sc as plsc

N = 50000
D = 64
NC = 2
NS = 16
LN = 16

HP = 25088
NP = NC * HP
DUMP = HP
TAB = HP + 256
STR_R = TAB // NS
STR_N = HP // NS
RCH = 112
NRCH = STR_N // RCH

EPT = 51200
EPAD = NS * EPT
BLK = 2048
NBLK = EPT // BLK
CH = 128
NCH = BLK // CH
LCAP = EPT + BLK

_mesh = plsc.VectorSubcoreMesh(
    core_axis_name="c", subcore_axis_name="s", num_cores=NC, num_subcores=NS
)


def _rsqrt16(x):
    i = lax.bitcast_convert_type(x, jnp.int32)
    i = 0x5F3759DF - lax.shift_right_logical(i, 1)
    y = lax.bitcast_convert_type(i, jnp.float32)
    for _ in range(3):
        y = y * (1.5 - 0.5 * x * y * y)
    return jnp.where(x > 0.5, y, 0.0)


def _prep_body(src_hbm, dst_hbm, e0_hbm,
               dinv_hbm, u0_hbm, srcl_hbm, locl_hbm, cnt_hbm,
               deg_s, dinv_s, src_sp, loc_sp, src_blk, dst_blk,
               idx_buf, ldl_buf, zpad, dpad, ones_v, cvec, pfx,
               vbuf, rbuf, sem):
    c = lax.axis_index("c")
    s = lax.axis_index("s")

    for v in range(STR_R // LN):
        vbuf[pl.ds(v * LN, LN)] = jnp.zeros((LN,), jnp.float32)
    pltpu.sync_copy(vbuf, deg_s.at[pl.ds(s * STR_R, STR_R)])
    for v in range(CH // LN):
        ones_v[pl.ds(v * LN, LN)] = jnp.ones((LN,), jnp.float32)
    plsc.subcore_barrier()

    lo = c * HP
    base = s * LCAP
    iota = lax.iota(jnp.int32, LN)
    for v in range(3):
        pfx[pl.ds(v * LN, LN)] = jnp.zeros((LN,), jnp.int32)

    def blk_body(b, pos):
        e_base = s * EPT + b * BLK
        pltpu.sync_copy(src_hbm.at[pl.ds(e_base, BLK)], src_blk)
        pltpu.sync_copy(dst_hbm.at[pl.ds(e_base, BLK)], dst_blk)

        descs = []
        for ch in range(NCH):
            if ch >= 4:
                for dsc in descs[3 * (ch - 4):3 * (ch - 3)]:
                    dsc.wait()

            def grp(g, pos, ch=ch):
                d = dst_blk[pl.ds(ch * CH + g * LN, LN)]
                dl = d - lo
                m = (dl >= 0) & (dl < HP)
                mi = jnp.where(m, jnp.int32(1), jnp.int32(0))
                x = mi
                for k in (1, 2, 4, 8):
                    pfx[pl.ds(LN, LN)] = x
                    x = x + pfx[pl.ds(LN - k, LN)]
                y = mi
                for k in (1, 2, 4, 8):
                    pfx[pl.ds(LN, LN)] = y
                    y = y + pfx[pl.ds(LN + k, LN)]
                tot = x + y - mi
                idx_buf[ch, pl.ds(g * LN, LN)] = jnp.where(
                    m, base + (pos + x - 1), base + (LCAP - 1))
                ldl_buf[ch, pl.ds(g * LN, LN)] = jnp.where(m, dl, DUMP)
                return pos + tot

            pos = lax.fori_loop(0, CH // LN, grp, pos)
            descs.append(pltpu.async_copy(
                src_blk.at[pl.ds(ch * CH, CH)],
                src_sp.at[idx_buf.at[ch]], sem))
            descs.append(pltpu.async_copy(
                ldl_buf.at[ch],
                loc_sp.at[idx_buf.at[ch]], sem))
            descs.append(pltpu.async_copy(
                ones_v, deg_s.at[ldl_buf.at[ch]], sem, add=True))
        for dsc in descs[3 * (NCH - 4):]:
            dsc.wait()
        return pos

    pos = lax.fori_loop(0, NBLK, blk_body, jnp.zeros((LN,), jnp.int32))
    cnt = pos[0]
    cntp_v = jnp.bitwise_and(pos + (LBLK - 1), ~jnp.int32(LBLK - 1))

    for v in range(CH // LN):
        zpad[pl.ds(v * LN, LN)] = jnp.zeros((LN,), jnp.int32)
        dpad[pl.ds(v * LN, LN)] = jnp.full((LN,), DUMP, jnp.int32)

    descs = []
    for j in range(NCH):
        if j >= 4:
            for dsc in descs[2 * (j - 4):2 * (j - 3)]:
                dsc.wait()
        for v in range(CH // LN):
            idx_buf[j, pl.ds(v * LN, LN)] = (
                base + cnt + (j * CH + v * LN) + iota)
        descs.append(pltpu.async_copy(
            zpad, src_sp.at[idx_buf.at[j]], sem))
        descs.append(pltpu.async_copy(
            dpad, loc_sp.at[idx_buf.at[j]], sem))
    for dsc in descs[2 * (NCH - 4):]:
        dsc.wait()

    cvec[pl.ds(0, LN)] = cntp_v
    pltpu.sync_copy(cvec, cnt_hbm.at[c, s])
    pltpu.sync_copy(src_sp.at[pl.ds(base, EPT)], srcl_hbm.at[c, s])
    pltpu.sync_copy(loc_sp.at[pl.ds(base, EPT)], locl_hbm.at[c, s])
    plsc.subcore_barrier()

    pltpu.sync_copy(deg_s.at[pl.ds(s * STR_R, STR_R)], vbuf)
    for v in range(STR_R // LN):
        deg = vbuf[pl.ds(v * LN, LN)]
        vbuf[pl.ds(v * LN, LN)] = _rsqrt16(deg)
    pltpu.sync_copy(vbuf, dinv_s.at[pl.ds(s * STR_R, STR_R)])
    plsc.subcore_barrier()

    r0 = s * STR_N
    g0 = c * HP + r0
    pltpu.sync_copy(dinv_s.at[pl.ds(r0, STR_N)], vbuf.at[pl.ds(0, STR_N)])
    pltpu.sync_copy(vbuf.at[pl.ds(0, STR_N)], dinv_hbm.at[pl.ds(g0, STR_N)])

    def wb_body(cc, _):
        pltpu.sync_copy(e0_hbm.at[pl.ds(g0 + cc * RCH, RCH)], rbuf)

        def grp_body(g, _):
            dvec = vbuf[pl.ds(cc * RCH + g * LN, LN)]
            for r in range(LN):
                di = dvec[r]
                row = g * LN + r
                for k in range(D // LN):
                    cs = pl.ds(k * LN, LN)
                    rbuf[row, cs] = rbuf[row, cs] * di
            return 0

        lax.fori_loop(0, RCH // LN, grp_body, 0)
        pltpu.sync_copy(rbuf, u0_hbm.at[pl.ds(g0 + cc * RCH, RCH)])
        return 0

    lax.fori_loop(0, NRCH, wb_body, 0)


KBUF = 3
GAH = 2
SCW = 1
LBLK = 1024
LNCH = LBLK // CH
LNBLK = EPT // LBLK


def _layer_body(first, last,
                u_hbm, srcl_hbm, locl_hbm, cnt_hbm, dinv_hbm, *rest):
    rest = list(rest)
    ssum_in = None if first else rest.pop(0)
    e0_hbm = rest.pop(0) if last else None
    if last:
        emean_hbm = rest.pop(0)
    else:
        unext_hbm = rest.pop(0)
        ssum_out = rest.pop(0)
    (acc_s, src_blk, loc_blk, cvec, rows, dvbuf, sem_g, sem_s, sem_e) = rest

    c = lax.axis_index("c")
    s = lax.axis_index("s")

    def zrow_body(r, _):
        for k in range(D // LN):
            rows[0, r, pl.ds(k * LN, LN)] = jnp.zeros((LN,), jnp.float32)
        return 0

    lax.fori_loop(0, CH, zrow_body, 0)

    z_d = [pltpu.async_copy(rows.at[0, pl.ds(0, RCH)],
                            acc_s.at[pl.ds(s * STR_N + cc * RCH, RCH)],
                            sem_e)
           for cc in range(NRCH)]
    for d in z_d:
        d.wait()

    @pl.when(s == 0)
    def _():
        pltpu.sync_copy(rows.at[0, pl.ds(0, 16)], acc_s.at[pl.ds(HP, 16)])

    pltpu.sync_copy(cnt_hbm.at[c, s], cvec)
    cntp = cvec[pl.ds(0, LN)][0]
    plsc.subcore_barrier()

    def eload(b, slot):
        pltpu.async_copy(srcl_hbm.at[c, s, pl.ds(b * LBLK, LBLK)],
                         src_blk.at[slot], sem_e)
        pltpu.async_copy(locl_hbm.at[c, s, pl.ds(b * LBLK, LBLK)],
                         loc_blk.at[slot], sem_e)

    def ewait(slot):
        pltpu.make_async_copy(srcl_hbm.at[c, s, pl.ds(0, LBLK)],
                              src_blk.at[slot], sem_e).wait()
        pltpu.make_async_copy(locl_hbm.at[c, s, pl.ds(0, LBLK)],
                              loc_blk.at[slot], sem_e).wait()

    def process(b, slot, bnext):
        ewait(slot)

        @pl.when(bnext * LBLK < cntp)
        def _():
            eload(bnext, 1 - slot)

        def gather(j):
            idx = src_blk.at[slot, pl.ds(j * CH, CH)]
            return pltpu.async_copy(u_hbm.at[idx], rows.at[j % KBUF], sem_g)

        def scatter(j):
            idx = loc_blk.at[slot, pl.ds(j * CH, CH)]
            return pltpu.async_copy(
                rows.at[j % KBUF], acc_s.at[idx], sem_s, add=True)

        g_d = [gather(j) for j in range(GAH)]
        s_d = []
        for j in range(LNCH):
            if j >= SCW:
                s_d[j - SCW].wait()
            if j + GAH < LNCH:
                g_d.append(gather(j + GAH))
            g_d[j].wait()
            s_d.append(scatter(j))
        for j in range(LNCH - SCW, LNCH):
            s_d[j].wait()

    @pl.when(0 < cntp)
    def _():
        eload(0, 0)

    def pair_body(i, _):
        b0 = 2 * i
        b1 = 2 * i + 1

        @pl.when(b0 * LBLK < cntp)
        def _():
            process(b0, 0, b1)

        @pl.when(b1 * LBLK < cntp)
        def _():
            process(b1, 1, b1 + 1)

        return 0

    lax.fori_loop(0, (LNBLK + 1) // 2, pair_body, 0)
    plsc.subcore_barrier()

    def wb_body(cc, _):
        r0 = s * STR_N + cc * RCH
        g0 = c * HP + r0
        pltpu.sync_copy(acc_s.at[pl.ds(r0, RCH)], rows.at[0, pl.ds(0, RCH)])
        pltpu.sync_copy(dinv_hbm.at[pl.ds(g0, RCH)], dvbuf.at[pl.ds(0, RCH)])
        if not first:
            pltpu.sync_copy(ssum_in.at[pl.ds(g0, RCH)],
                            rows.at[1, pl.ds(0, RCH)])
        if last:
            pltpu.sync_copy(e0_hbm.at[pl.ds(g0, RCH)],
                            rows.at[2, pl.ds(0, RCH)])

        def grp_body(g, _):
            dvec = dvbuf[pl.ds(g * LN, LN)]
            for r in range(LN):
                di = dvec[r]
                row = g * LN + r
                for k in range(D // LN):
                    cs = pl.ds(k * LN, LN)
                    sv = rows[0, row, cs]
                    ss = sv if first else rows[1, row, cs] + sv
                    if last:
                        rows[2, row, cs] = (rows[2, row, cs] + di * ss) * 0.25
                    else:
                        rows[1, row, cs] = ss
                        rows[0, row, cs] = sv * (di * di)
            return 0

        lax.fori_loop(0, RCH // LN, grp_body, 0)
        if last:
            pltpu.sync_copy(rows.at[2, pl.ds(0, RCH)],
                            emean_hbm.at[pl.ds(g0, RCH)])
        else:
            pltpu.sync_copy(rows.at[0, pl.ds(0, RCH)],
                            unext_hbm.at[pl.ds(g0, RCH)])
            pltpu.sync_copy(rows.at[1, pl.ds(0, RCH)],
                            ssum_out.at[pl.ds(g0, RCH)])
        return 0

    lax.fori_loop(0, NRCH, wb_body, 0)


def _f32(*shape):
    return jax.ShapeDtypeStruct(shape, jnp.float32)


def _i32(*shape):
    return jax.ShapeDtypeStruct(shape, jnp.int32)


_params = pltpu.CompilerParams(use_tc_tiling_on_sc=False)

_prep = pl.kernel(
    _prep_body,
    out_type=(
        _f32(NP),
        _f32(NP, D),
        _i32(NC, NS, EPT),
        _i32(NC, NS, EPT),
        _i32(NC, NS, LN),
    ),
    mesh=_mesh,
    compiler_params=_params,
    scratch_types=[
        pltpu.VMEM_SHARED((TAB,), jnp.float32),
        pltpu.VMEM_SHARED((TAB,), jnp.float32),
        pltpu.VMEM_SHARED((NS * LCAP,), jnp.int32),
        pltpu.VMEM_SHARED((NS * LCAP,), jnp.int32),
        pltpu.VMEM((BLK,), jnp.int32),
        pltpu.VMEM((BLK,), jnp.int32),
        pltpu.VMEM((NCH, CH), jnp.int32),
        pltpu.VMEM((NCH, CH), jnp.int32),
        pltpu.VMEM((CH,), jnp.int32),
        pltpu.VMEM((CH,), jnp.int32),
        pltpu.VMEM((CH,), jnp.float32),
        pltpu.VMEM((LN,), jnp.int32),
        pltpu.VMEM((3 * LN,), jnp.int32),
        pltpu.VMEM((STR_R,), jnp.float32),
        pltpu.VMEM((RCH, D), jnp.float32),
        pltpu.SemaphoreType.DMA,
    ],
)


def _make_layer(first, last):
    out_type = (_f32(NP, D),) if last else (_f32(NP, D), _f32(NP, D))
    return pl.kernel(
        functools.partial(_layer_body, first, last),
        out_type=out_type,
        mesh=_mesh,
        compiler_params=_params,
        scratch_types=[
            pltpu.VMEM_SHARED((HP + 16, D), jnp.float32),
            pltpu.VMEM((2, LBLK), jnp.int32),
            pltpu.VMEM((2, LBLK), jnp.int32),
            pltpu.VMEM((LN,), jnp.int32),
            pltpu.VMEM((KBUF, CH, D), jnp.float32),
            pltpu.VMEM((CH,), jnp.float32),
            pltpu.SemaphoreType.DMA,
            pltpu.SemaphoreType.DMA,
            pltpu.SemaphoreType.DMA,
        ],
    )


_layer_first = _make_layer(True, False)
_layer_mid = _make_layer(False, False)
_layer_last = _make_layer(False, True)


def kernel(x, edge_index, E0):
    src = edge_index[0]
    dst = edge_index[1]
    e = src.shape[0]
    srcp = jnp.concatenate([src, jnp.zeros((EPAD - e,), jnp.int32)])
    dstp = jnp.concatenate([dst, jnp.full((EPAD - e,), -1, jnp.int32)])
    e0p = jnp.pad(E0, ((0, NP - N), (0, 0)))

    dinv, u0, srcl, locl, cnt = _prep(srcp, dstp, e0p)
    u1, ss1 = _layer_first(u0, srcl, locl, cnt, dinv)
    u2, ss2 = _layer_mid(u1, srcl, locl, cnt, dinv, ss1)
    (emean_p,) = _layer_last(u2, srcl, locl, cnt, dinv, ss2, e0p)
    return (E0, emean_p[:N])
